# Initial kernel scaffold; baseline (speedup 1.0000x reference)
import jax
import jax.numpy as jnp
from jax.experimental import pallas as pl


def kernel(x, edge_index, W1, b1, W2, b2, gamma, beta):
    raise NotImplementedError("write your pallas kernel here")


# speedup vs baseline: 15.3550x; 15.3550x over previous
import jax
import jax.numpy as jnp
from jax import lax
from jax.experimental import pallas as pl
from jax.experimental.pallas import tpu as pltpu
from jax.experimental.pallas import tpu_sc as plsc

N = 10000
D = 128
E = 320000
BN_EPS = 1e-5

NC = 2
NS = 16
NW = NC * NS

N_PAD = 10240
PAD_LO = N
PAD_SZ = N_PAD - N

WIN = 128
EPW = 10240
WPW = EPW // WIN
E_PAD = EPW * NW
RPT = N_PAD // NS

_mesh = plsc.VectorSubcoreMesh(
    core_axis_name="c", subcore_axis_name="s", num_cores=NC, num_subcores=NS)


def _sc_pre_body(src_hbm, dst_hbm, srcm_hbm, dstm_hbm, degp_hbm,
                 s_buf, d_buf, sm_buf, dm_buf, ones_buf, z_buf, deg_sh):
    cid = lax.axis_index("c")
    sid = lax.axis_index("s")
    wid = cid * NS + sid
    ebase = wid * EPW

    zeros16 = jnp.zeros((16,), jnp.float32)
    for i in range(RPT // 16):
        z_buf[pl.ds(i * 16, 16)] = zeros16
    for i in range(WIN // 16):
        ones_buf[pl.ds(i * 16, 16)] = jnp.ones((16,), jnp.float32)
    pltpu.sync_copy(z_buf, deg_sh.at[pl.ds(sid * RPT, RPT)])
    plsc.subcore_barrier()

    iota16 = lax.iota(jnp.int32, 16)

    def body(k, _):
        base = ebase + k * WIN
        pltpu.sync_copy(src_hbm.at[pl.ds(base, WIN)], s_buf)
        pltpu.sync_copy(dst_hbm.at[pl.ds(base, WIN)], d_buf)
        for j in range(WIN // 16):
            s = s_buf[pl.ds(j * 16, 16)]
            d = d_buf[pl.ds(j * 16, 16)]
            ge = (base + j * 16) + iota16
            rep = PAD_LO + lax.rem(ge, PAD_SZ)
            msk = s == d
            sm_buf[0, pl.ds(j * 16, 16)] = jnp.where(msk, rep, s)
            dm_buf[0, pl.ds(j * 16, 16)] = jnp.where(msk, rep, d)
        pltpu.sync_copy(sm_buf.at[0], srcm_hbm.at[pl.ds(base, WIN)])
        pltpu.sync_copy(dm_buf.at[0], dstm_hbm.at[pl.ds(base, WIN)])
        pltpu.sync_copy(ones_buf, deg_sh.at[dm_buf.at[0]], add=True)
        return 0

    lax.fori_loop(0, WPW, body, 0)
    plsc.subcore_barrier()
    pltpu.sync_copy(deg_sh.at[pl.ds(sid * RPT, RPT)],
                    degp_hbm.at[cid, pl.ds(sid * RPT, RPT)])


_sc_pre = pl.kernel(
    _sc_pre_body,
    out_type=(
        jax.ShapeDtypeStruct((E_PAD,), jnp.int32),
        jax.ShapeDtypeStruct((E_PAD,), jnp.int32),
        jax.ShapeDtypeStruct((NC, N_PAD), jnp.float32),
    ),
    mesh=_mesh,
    scratch_types=[
        pltpu.VMEM((WIN,), jnp.int32),
        pltpu.VMEM((WIN,), jnp.int32),
        pltpu.VMEM((1, WIN), jnp.int32),
        pltpu.VMEM((1, WIN), jnp.int32),
        pltpu.VMEM((WIN,), jnp.float32),
        pltpu.VMEM((RPT,), jnp.float32),
        pltpu.VMEM_SHARED((N_PAD,), jnp.float32),
    ],
)


def _sc_spmm_body(hp_hbm, dstm_hbm, srcm_hbm, accp_hbm,
                  idx_d, idx_s, rows, acc_sh):
    cid = lax.axis_index("c")
    sid = lax.axis_index("s")
    wid = cid * NS + sid
    ebase = wid * EPW

    pltpu.sync_copy(hp_hbm.at[pl.ds(sid * RPT, RPT)],
                    acc_sh.at[pl.ds(sid * RPT, RPT)])
    plsc.subcore_barrier()

    def body(k, _):
        base = ebase + k * WIN
        pltpu.sync_copy(dstm_hbm.at[pl.ds(base, WIN)], idx_d.at[0])
        pltpu.sync_copy(hp_hbm.at[idx_d.at[0]], rows.at[0])
        pltpu.sync_copy(srcm_hbm.at[pl.ds(base, WIN)], idx_s.at[0])
        pltpu.sync_copy(rows.at[0], acc_sh.at[idx_s.at[0]], add=True)
        return 0

    lax.fori_loop(0, WPW, body, 0)
    plsc.subcore_barrier()
    pltpu.sync_copy(acc_sh.at[pl.ds(sid * RPT, RPT)],
                    accp_hbm.at[cid, pl.ds(sid * RPT, RPT)])


_sc_spmm = pl.kernel(
    _sc_spmm_body,
    out_type=jax.ShapeDtypeStruct((NC, N_PAD, D), jnp.float32),
    mesh=_mesh,
    scratch_types=[
        pltpu.VMEM((2, WIN), jnp.int32),
        pltpu.VMEM((2, WIN), jnp.int32),
        pltpu.VMEM((2, WIN, D), jnp.float32),
        pltpu.VMEM_SHARED((N_PAD, D), jnp.float32),
    ],
)


def _dinv_from(degp):
    return lax.rsqrt(degp[0] + degp[1] + 1.0)


def _tc1_body(x_ref, w1_ref, b1_ref, degp_ref, out_ref):
    dinv = _dinv_from(degp_ref[...])
    h = jnp.dot(x_ref[...], w1_ref[...],
                preferred_element_type=jnp.float32) + b1_ref[...]
    out_ref[:N] = h * dinv[:N, None]
    out_ref[N:] = jnp.zeros((N_PAD - N, D), jnp.float32)


def _tc2_body(acc_ref, hp_ref, degp_ref, w2_ref, b2_ref, g_ref, be_ref,
              out_ref):
    dinv = _dinv_from(degp_ref[...])[:N, None]
    s = (acc_ref[0, :N] + acc_ref[1, :N] - hp_ref[:N]) * dinv
    mean = jnp.mean(s, axis=0)
    var = jnp.mean(jnp.square(s), axis=0) - jnp.square(mean)
    sn = (s - mean) * lax.rsqrt(var + BN_EPS) * g_ref[...] + be_ref[...]
    r = jnp.maximum(sn, 0.0)
    h2 = jnp.dot(r, w2_ref[...],
                 preferred_element_type=jnp.float32) + b2_ref[...]
    out_ref[:N] = h2 * dinv
    out_ref[N:] = jnp.zeros((N_PAD - N, D), jnp.float32)


def _tc3_body(acc_ref, hp_ref, degp_ref, out_ref):
    dinv = _dinv_from(degp_ref[...])[:N, None]
    out_ref[...] = (acc_ref[0, :N] + acc_ref[1, :N] - hp_ref[:N]) * dinv


_tc1 = pl.pallas_call(
    _tc1_body, out_shape=jax.ShapeDtypeStruct((N_PAD, D), jnp.float32))
_tc2 = pl.pallas_call(
    _tc2_body, out_shape=jax.ShapeDtypeStruct((N_PAD, D), jnp.float32))
_tc3 = pl.pallas_call(
    _tc3_body, out_shape=jax.ShapeDtypeStruct((N, D), jnp.float32))


def kernel(x, edge_index, W1, b1, W2, b2, gamma, beta):
    pad = jnp.zeros((E_PAD - E,), jnp.int32)
    src_in = jnp.concatenate([edge_index[0], pad])
    dst_in = jnp.concatenate([edge_index[1], pad])
    srcm, dstm, degp = _sc_pre(src_in, dst_in)
    h1p = _tc1(x, W1, b1, degp)
    acc1 = _sc_spmm(h1p, dstm, srcm)
    h2p = _tc2(acc1, h1p, degp, W2, b2, gamma, beta)
    acc2 = _sc_spmm(h2p, dstm, srcm)
    return _tc3(acc2, h2p, degp)


# speedup vs baseline: 35.0864x; 2.2850x over previous
import jax
import jax.numpy as jnp
from jax import lax
from jax.experimental import pallas as pl
from jax.experimental.pallas import tpu as pltpu
from jax.experimental.pallas import tpu_sc as plsc

N = 10000
D = 128
E = 320000
BN_EPS = 1e-5

NC = 2
NS = 16
NW = NC * NS

N_PAD = 10112
PAD_LO = N
PAD_MSK = 63

WIN = 128
EPW = 10240
WPW = EPW // WIN
E_PAD = EPW * NW
NWIN = E_PAD // WIN
RPT = N_PAD // NS
N_PADD = 10240
RPTD = N_PADD // NS
CH = 16
NCH = WPW // CH

_mesh = plsc.VectorSubcoreMesh(
    core_axis_name="c", subcore_axis_name="s", num_cores=NC, num_subcores=NS)


def _sc_pre_body(src_hbm, dst_hbm, srcm_hbm, dstm_hbm, degp_hbm,
                 s_buf, d_buf, sm_buf, dm_buf, ones_buf, z_buf,
                 sem_i, sem_o, deg_sh):
    cid = lax.axis_index("c")
    sid = lax.axis_index("s")
    wid = cid * NS + sid
    wrow = pl.multiple_of(wid * WPW, WPW)

    ld_s = pltpu.async_copy(src_hbm.at[pl.ds(wrow, WPW)], s_buf, sem_i)
    ld_d = pltpu.async_copy(dst_hbm.at[pl.ds(wrow, WPW)], d_buf, sem_i)
    zeros16 = jnp.zeros((16,), jnp.float32)
    for i in range(640 // 16):
        z_buf[pl.ds(i * 16, 16)] = zeros16
    for i in range(WIN // 16):
        ones_buf[pl.ds(i * 16, 16)] = jnp.ones((16,), jnp.float32)
    pltpu.sync_copy(z_buf, deg_sh.at[pl.ds(pl.multiple_of(sid * RPTD, 16),
                                           RPTD)])
    ld_s.wait()
    ld_d.wait()

    iota16 = lax.iota(jnp.int32, 16)

    def mask_body(k, _):
        base = (wrow + k) * WIN
        for j in range(WIN // 16):
            s = s_buf[k, pl.ds(j * 16, 16)]
            d = d_buf[k, pl.ds(j * 16, 16)]
            ge = (base + j * 16) + iota16
            rep = PAD_LO + (ge & PAD_MSK)
            msk = s == d
            sm_buf[k, pl.ds(j * 16, 16)] = jnp.where(msk, rep, s)
            dm_buf[k, pl.ds(j * 16, 16)] = jnp.where(msk, rep, d)
        return 0

    lax.fori_loop(0, WPW, mask_body, 0)
    out_s = pltpu.async_copy(sm_buf, srcm_hbm.at[pl.ds(wrow, WPW)], sem_o)
    out_d = pltpu.async_copy(dm_buf, dstm_hbm.at[pl.ds(wrow, WPW)], sem_o)
    plsc.subcore_barrier()

    def hist_body(g, _):
        descs = [pltpu.async_copy(ones_buf, deg_sh.at[dm_buf.at[4 * g + b]],
                                  sem_i, add=True) for b in range(4)]
        for desc in descs:
            desc.wait()
        return 0

    lax.fori_loop(0, WPW // 4, hist_body, 0)
    out_s.wait()
    out_d.wait()
    plsc.subcore_barrier()
    dlo = pl.multiple_of(sid * RPTD, 16)
    pltpu.sync_copy(deg_sh.at[pl.ds(dlo, RPTD)],
                    degp_hbm.at[pl.ds(pl.multiple_of(cid * N_PADD + sid * RPTD,
                                                     16), RPTD)])


_sc_pre = pl.kernel(
    _sc_pre_body,
    out_type=(
        jax.ShapeDtypeStruct((NWIN, WIN), jnp.int32),
        jax.ShapeDtypeStruct((NWIN, WIN), jnp.int32),
        jax.ShapeDtypeStruct((NC * N_PADD,), jnp.float32),
    ),
    mesh=_mesh,
    scratch_types=[
        pltpu.VMEM((WPW, WIN), jnp.int32),
        pltpu.VMEM((WPW, WIN), jnp.int32),
        pltpu.VMEM((WPW, WIN), jnp.int32),
        pltpu.VMEM((WPW, WIN), jnp.int32),
        pltpu.VMEM((WIN,), jnp.float32),
        pltpu.VMEM((640,), jnp.float32),
        pltpu.SemaphoreType.DMA,
        pltpu.SemaphoreType.DMA,
        pltpu.VMEM_SHARED((N_PADD,), jnp.float32),
    ],
)


def _sc_spmm_body(hp_hbm, dstm_hbm, srcm_hbm, accp_hbm,
                  idx_d, idx_s, rows, sem_i, sem0, sem1, acc_sh):
    cid = lax.axis_index("c")
    sid = lax.axis_index("s")
    wid = cid * NS + sid
    wrow = pl.multiple_of(wid * WPW, WPW)
    sems = (sem0, sem1)

    def stage(c):
        cb = c % 2
        return (
            pltpu.async_copy(
                dstm_hbm.at[pl.ds(pl.multiple_of(wrow + c * CH, CH), CH)],
                idx_d.at[cb], sem_i),
            pltpu.async_copy(
                srcm_hbm.at[pl.ds(pl.multiple_of(wrow + c * CH, CH), CH)],
                idx_s.at[cb], sem_i),
        )

    def gather(k):
        return pltpu.async_copy(hp_hbm.at[idx_d.at[(k // CH) % 2, k % CH]],
                                rows.at[k % 2], sems[k % 2])

    st = {0: stage(0)}
    rlo = pl.multiple_of(sid * RPT, 8)
    pltpu.sync_copy(hp_hbm.at[pl.ds(rlo, RPT)], acc_sh.at[pl.ds(rlo, RPT)])
    for d in st[0]:
        d.wait()
    plsc.subcore_barrier()

    gd = gather(0)
    for k in range(WPW):
        c = k // CH
        if k % CH == 0 and c + 1 < NCH:
            st[c + 1] = stage(c + 1)
        gnext = None
        if k + 1 < WPW:
            if (k + 1) % CH == 0:
                for d in st[(k + 1) // CH]:
                    d.wait()
            gnext = gather(k + 1)
        gd.wait()
        pltpu.sync_copy(rows.at[k % 2],
                        acc_sh.at[idx_s.at[(k // CH) % 2, k % CH]], add=True)
        gd = gnext

    plsc.subcore_barrier()
    pltpu.sync_copy(acc_sh.at[pl.ds(rlo, RPT)],
                    accp_hbm.at[pl.ds(pl.multiple_of(cid * N_PAD + sid * RPT, 8),
                                      RPT)])


_sc_spmm = pl.kernel(
    _sc_spmm_body,
    out_type=jax.ShapeDtypeStruct((NC * N_PAD, D), jnp.float32),
    mesh=_mesh,
    scratch_types=[
        pltpu.VMEM((2, CH, WIN), jnp.int32),
        pltpu.VMEM((2, CH, WIN), jnp.int32),
        pltpu.VMEM((2, WIN, D), jnp.float32),
        pltpu.SemaphoreType.DMA,
        pltpu.SemaphoreType.DMA,
        pltpu.SemaphoreType.DMA,
        pltpu.VMEM_SHARED((N_PAD, D), jnp.float32),
    ],
)


def _dinv_from(degp):
    return lax.rsqrt(degp[0] + degp[1] + 1.0)


def _tc1_body(x_ref, w1_ref, b1_ref, degp_ref, out_ref):
    dinv = _dinv_from(degp_ref[...])
    h = jnp.dot(x_ref[...], w1_ref[...],
                preferred_element_type=jnp.float32) + b1_ref[...]
    out_ref[:N] = h * dinv[:N, None]
    out_ref[N:] = jnp.zeros((N_PAD - N, D), jnp.float32)


def _tc2_body(acc_ref, hp_ref, degp_ref, w2_ref, b2_ref, g_ref, be_ref,
              out_ref):
    dinv = _dinv_from(degp_ref[...])[:N, None]
    s = (acc_ref[0, :N] + acc_ref[1, :N] - hp_ref[:N]) * dinv
    mean = jnp.mean(s, axis=0)
    var = jnp.mean(jnp.square(s), axis=0) - jnp.square(mean)
    sn = (s - mean) * lax.rsqrt(var + BN_EPS) * g_ref[...] + be_ref[...]
    r = jnp.maximum(sn, 0.0)
    h2 = jnp.dot(r, w2_ref[...],
                 preferred_element_type=jnp.float32) + b2_ref[...]
    out_ref[:N] = h2 * dinv
    out_ref[N:] = jnp.zeros((N_PAD - N, D), jnp.float32)


def _tc3_body(acc_ref, hp_ref, degp_ref, out_ref):
    dinv = _dinv_from(degp_ref[...])[:N, None]
    out_ref[...] = (acc_ref[0, :N] + acc_ref[1, :N] - hp_ref[:N]) * dinv


_tc1 = pl.pallas_call(
    _tc1_body, out_shape=jax.ShapeDtypeStruct((N_PAD, D), jnp.float32))
_tc2 = pl.pallas_call(
    _tc2_body, out_shape=jax.ShapeDtypeStruct((N_PAD, D), jnp.float32))
_tc3 = pl.pallas_call(
    _tc3_body, out_shape=jax.ShapeDtypeStruct((N, D), jnp.float32))


def kernel(x, edge_index, W1, b1, W2, b2, gamma, beta):
    pad = jnp.zeros((E_PAD - E,), jnp.int32)
    src_in = jnp.concatenate([edge_index[0], pad]).reshape(NWIN, WIN)
    dst_in = jnp.concatenate([edge_index[1], pad]).reshape(NWIN, WIN)
    srcm, dstm, degp = _sc_pre(src_in, dst_in)
    degp = degp.reshape(NC, N_PADD)
    h1p = _tc1(x, W1, b1, degp)
    acc1 = _sc_spmm(h1p, dstm, srcm).reshape(NC, N_PAD, D)
    h2p = _tc2(acc1, h1p, degp, W2, b2, gamma, beta)
    acc2 = _sc_spmm(h2p, dstm, srcm).reshape(NC, N_PAD, D)
    return _tc3(acc2, h2p, degp)


# speedup vs baseline: 35.8444x; 1.0216x over previous
import jax
import jax.numpy as jnp
from jax import lax
from jax.experimental import pallas as pl
from jax.experimental.pallas import tpu as pltpu
from jax.experimental.pallas import tpu_sc as plsc

N = 10000
D = 128
E = 320000
BN_EPS = 1e-5

NC = 2
NS = 16
NW = NC * NS

N_PAD = 10112
PAD_LO = N
PAD_MSK = 63

WIN = 128
EPW = 10240
WPW = EPW // WIN
E_PAD = EPW * NW
NWIN = E_PAD // WIN
RPT = N_PAD // NS
N_PADD = 10240
RPTD = N_PADD // NS
CH = 16
NCH = WPW // CH

_mesh = plsc.VectorSubcoreMesh(
    core_axis_name="c", subcore_axis_name="s", num_cores=NC, num_subcores=NS)


def _sc_pre_body(src_hbm, dst_hbm, srcm_hbm, dstm_hbm, degp_hbm,
                 s_buf, d_buf, sm_buf, dm_buf, ones_buf, z_buf,
                 sem_i, sem_o, deg_sh):
    cid = lax.axis_index("c")
    sid = lax.axis_index("s")
    wid = cid * NS + sid
    wrow = pl.multiple_of(wid * WPW, WPW)

    ld_s = pltpu.async_copy(src_hbm.at[pl.ds(wrow, WPW)], s_buf, sem_i)
    ld_d = pltpu.async_copy(dst_hbm.at[pl.ds(wrow, WPW)], d_buf, sem_i)
    zeros16 = jnp.zeros((16,), jnp.float32)
    for i in range(640 // 16):
        z_buf[pl.ds(i * 16, 16)] = zeros16
    for i in range(WIN // 16):
        ones_buf[pl.ds(i * 16, 16)] = jnp.ones((16,), jnp.float32)
    pltpu.sync_copy(z_buf, deg_sh.at[pl.ds(pl.multiple_of(sid * RPTD, 16),
                                           RPTD)])
    ld_s.wait()
    ld_d.wait()

    iota16 = lax.iota(jnp.int32, 16)

    def mask_body(k, _):
        base = (wrow + k) * WIN
        for j in range(WIN // 16):
            s = s_buf[k, pl.ds(j * 16, 16)]
            d = d_buf[k, pl.ds(j * 16, 16)]
            ge = (base + j * 16) + iota16
            rep = PAD_LO + (ge & PAD_MSK)
            msk = s == d
            sm_buf[k, pl.ds(j * 16, 16)] = jnp.where(msk, rep, s)
            dm_buf[k, pl.ds(j * 16, 16)] = jnp.where(msk, rep, d)
        return 0

    lax.fori_loop(0, WPW, mask_body, 0)
    out_s = pltpu.async_copy(sm_buf, srcm_hbm.at[pl.ds(wrow, WPW)], sem_o)
    out_d = pltpu.async_copy(dm_buf, dstm_hbm.at[pl.ds(wrow, WPW)], sem_o)
    plsc.subcore_barrier()

    def hist_body(g, _):
        descs = [pltpu.async_copy(ones_buf, deg_sh.at[dm_buf.at[4 * g + b]],
                                  sem_i, add=True) for b in range(4)]
        for desc in descs:
            desc.wait()
        return 0

    lax.fori_loop(0, WPW // 4, hist_body, 0)
    out_s.wait()
    out_d.wait()
    plsc.subcore_barrier()
    dlo = pl.multiple_of(sid * RPTD, 16)
    pltpu.sync_copy(deg_sh.at[pl.ds(dlo, RPTD)],
                    degp_hbm.at[pl.ds(pl.multiple_of(cid * N_PADD + sid * RPTD,
                                                     16), RPTD)])


_sc_pre = pl.kernel(
    _sc_pre_body,
    out_type=(
        jax.ShapeDtypeStruct((NWIN, WIN), jnp.int32),
        jax.ShapeDtypeStruct((NWIN, WIN), jnp.int32),
        jax.ShapeDtypeStruct((NC * N_PADD,), jnp.float32),
    ),
    mesh=_mesh,
    scratch_types=[
        pltpu.VMEM((WPW, WIN), jnp.int32),
        pltpu.VMEM((WPW, WIN), jnp.int32),
        pltpu.VMEM((WPW, WIN), jnp.int32),
        pltpu.VMEM((WPW, WIN), jnp.int32),
        pltpu.VMEM((WIN,), jnp.float32),
        pltpu.VMEM((640,), jnp.float32),
        pltpu.SemaphoreType.DMA,
        pltpu.SemaphoreType.DMA,
        pltpu.VMEM_SHARED((N_PADD,), jnp.float32),
    ],
)


def _sc_spmm_body(hp_hbm, dstm_hbm, srcm_hbm, accp_hbm,
                  idx_d, idx_s, rows, sem_i, sem0, sem1, sem_a,
                  sem_s0, sem_s1, acc_sh):
    cid = lax.axis_index("c")
    sid = lax.axis_index("s")
    wid = cid * NS + sid
    wrow = pl.multiple_of(wid * WPW, WPW)
    sems = (sem0, sem1)

    def stage(c):
        cb = c % 2
        return (
            pltpu.async_copy(
                dstm_hbm.at[pl.ds(pl.multiple_of(wrow + c * CH, CH), CH)],
                idx_d.at[cb], sem_i),
            pltpu.async_copy(
                srcm_hbm.at[pl.ds(pl.multiple_of(wrow + c * CH, CH), CH)],
                idx_s.at[cb], sem_i),
        )

    def gather(k):
        return pltpu.async_copy(hp_hbm.at[idx_d.at[(k // CH) % 2, k % CH]],
                                rows.at[k % 2], sems[k % 2])

    st = {0: stage(0)}
    rlo = pl.multiple_of(sid * RPT, 8)
    init = pltpu.async_copy(hp_hbm.at[pl.ds(rlo, RPT)],
                            acc_sh.at[pl.ds(rlo, RPT)], sem_a)
    for d in st[0]:
        d.wait()

    ssems = (sem_s0, sem_s1)
    sd = {}

    def scatter(k):
        return pltpu.async_copy(rows.at[k % 2],
                                acc_sh.at[idx_s.at[(k // CH) % 2, k % CH]],
                                ssems[k % 2], add=True)

    gd = gather(0)
    for k in range(WPW):
        c = k // CH
        if k % CH == 0 and c + 1 < NCH:
            st[c + 1] = stage(c + 1)
        gnext = None
        if k + 1 < WPW:
            if (k + 1) % CH == 0:
                for d in st[(k + 1) // CH]:
                    d.wait()
            if k - 1 >= 0:
                sd[(k + 1) % 2].wait()
            gnext = gather(k + 1)
        gd.wait()
        if k == 0:
            init.wait()
            plsc.subcore_barrier()
        sd[k % 2] = scatter(k)
        gd = gnext

    sd[(WPW - 2) % 2].wait()
    sd[(WPW - 1) % 2].wait()
    plsc.subcore_barrier()
    pltpu.sync_copy(acc_sh.at[pl.ds(rlo, RPT)],
                    accp_hbm.at[pl.ds(pl.multiple_of(cid * N_PAD + sid * RPT, 8),
                                      RPT)])


_sc_spmm = pl.kernel(
    _sc_spmm_body,
    out_type=jax.ShapeDtypeStruct((NC * N_PAD, D), jnp.float32),
    mesh=_mesh,
    scratch_types=[
        pltpu.VMEM((2, CH, WIN), jnp.int32),
        pltpu.VMEM((2, CH, WIN), jnp.int32),
        pltpu.VMEM((2, WIN, D), jnp.float32),
        pltpu.SemaphoreType.DMA,
        pltpu.SemaphoreType.DMA,
        pltpu.SemaphoreType.DMA,
        pltpu.SemaphoreType.DMA,
        pltpu.SemaphoreType.DMA,
        pltpu.SemaphoreType.DMA,
        pltpu.VMEM_SHARED((N_PAD, D), jnp.float32),
    ],
)


def _dinv_from(degp):
    return lax.rsqrt(degp[0] + degp[1] + 1.0)


def _tc1_body(x_ref, w1_ref, b1_ref, degp_ref, out_ref):
    dinv = _dinv_from(degp_ref[...])
    h = jnp.dot(x_ref[...], w1_ref[...],
                preferred_element_type=jnp.float32) + b1_ref[...]
    out_ref[:N] = h * dinv[:N, None]
    out_ref[N:] = jnp.zeros((N_PAD - N, D), jnp.float32)


def _tc2_body(acc_ref, hp_ref, degp_ref, w2_ref, b2_ref, g_ref, be_ref,
              out_ref):
    dinv = _dinv_from(degp_ref[...])[:N, None]
    s = (acc_ref[0, :N] + acc_ref[1, :N] - hp_ref[:N]) * dinv
    mean = jnp.mean(s, axis=0)
    var = jnp.mean(jnp.square(s), axis=0) - jnp.square(mean)
    sn = (s - mean) * lax.rsqrt(var + BN_EPS) * g_ref[...] + be_ref[...]
    r = jnp.maximum(sn, 0.0)
    h2 = jnp.dot(r, w2_ref[...],
                 preferred_element_type=jnp.float32) + b2_ref[...]
    out_ref[:N] = h2 * dinv
    out_ref[N:] = jnp.zeros((N_PAD - N, D), jnp.float32)


def _tc3_body(acc_ref, hp_ref, degp_ref, out_ref):
    dinv = _dinv_from(degp_ref[...])[:N, None]
    out_ref[...] = (acc_ref[0, :N] + acc_ref[1, :N] - hp_ref[:N]) * dinv


_tc1 = pl.pallas_call(
    _tc1_body, out_shape=jax.ShapeDtypeStruct((N_PAD, D), jnp.float32))
_tc2 = pl.pallas_call(
    _tc2_body, out_shape=jax.ShapeDtypeStruct((N_PAD, D), jnp.float32))
_tc3 = pl.pallas_call(
    _tc3_body, out_shape=jax.ShapeDtypeStruct((N, D), jnp.float32))


def kernel(x, edge_index, W1, b1, W2, b2, gamma, beta):
    pad = jnp.zeros((E_PAD - E,), jnp.int32)
    src_in = jnp.concatenate([edge_index[0], pad]).reshape(NWIN, WIN)
    dst_in = jnp.concatenate([edge_index[1], pad]).reshape(NWIN, WIN)
    srcm, dstm, degp = _sc_pre(src_in, dst_in)
    degp = degp.reshape(NC, N_PADD)
    h1p = _tc1(x, W1, b1, degp)
    acc1 = _sc_spmm(h1p, dstm, srcm).reshape(NC, N_PAD, D)
    h2p = _tc2(acc1, h1p, degp, W2, b2, gamma, beta)
    acc2 = _sc_spmm(h2p, dstm, srcm).reshape(NC, N_PAD, D)
    return _tc3(acc2, h2p, degp)


# speedup vs baseline: 36.0555x; 1.0059x over previous
import jax
import jax.numpy as jnp
from jax import lax
from jax.experimental import pallas as pl
from jax.experimental.pallas import tpu as pltpu
from jax.experimental.pallas import tpu_sc as plsc

N = 10000
D = 128
E = 320000
BN_EPS = 1e-5

NC = 2
NS = 16
NW = NC * NS

N_PAD = 10112
PAD_LO = N
PAD_MSK = 63

WIN = 128
EPW = 10240
WPW = EPW // WIN
E_PAD = EPW * NW
NWIN = E_PAD // WIN
RPT = N_PAD // NS
N_PADD = 10240
RPTD = N_PADD // NS
CH = 16
NCH = WPW // CH

_mesh = plsc.VectorSubcoreMesh(
    core_axis_name="c", subcore_axis_name="s", num_cores=NC, num_subcores=NS)


def _sc_pre_body(src_hbm, dst_hbm, srcm_hbm, dstm_hbm, degp_hbm,
                 s_buf, d_buf, sm_buf, dm_buf, ones_buf, z_buf,
                 sem_i, sem_o, deg_sh):
    cid = lax.axis_index("c")
    sid = lax.axis_index("s")
    wid = cid * NS + sid
    wrow = pl.multiple_of(wid * WPW, WPW)

    ld_s = pltpu.async_copy(src_hbm.at[pl.ds(wrow, WPW)], s_buf, sem_i)
    ld_d = pltpu.async_copy(dst_hbm.at[pl.ds(wrow, WPW)], d_buf, sem_i)
    zeros16 = jnp.zeros((16,), jnp.float32)
    for i in range(640 // 16):
        z_buf[pl.ds(i * 16, 16)] = zeros16
    for i in range(WIN // 16):
        ones_buf[pl.ds(i * 16, 16)] = jnp.ones((16,), jnp.float32)
    pltpu.sync_copy(z_buf, deg_sh.at[pl.ds(pl.multiple_of(sid * RPTD, 16),
                                           RPTD)])
    ld_s.wait()
    ld_d.wait()

    iota16 = lax.iota(jnp.int32, 16)

    def mask_body(k, _):
        base = (wrow + k) * WIN
        for j in range(WIN // 16):
            s = s_buf[k, pl.ds(j * 16, 16)]
            d = d_buf[k, pl.ds(j * 16, 16)]
            ge = (base + j * 16) + iota16
            rep = PAD_LO + (ge & PAD_MSK)
            msk = s == d
            sm_buf[k, pl.ds(j * 16, 16)] = jnp.where(msk, rep, s)
            dm_buf[k, pl.ds(j * 16, 16)] = jnp.where(msk, rep, d)
        return 0

    lax.fori_loop(0, WPW, mask_body, 0)
    out_s = pltpu.async_copy(sm_buf, srcm_hbm.at[pl.ds(wrow, WPW)], sem_o)
    out_d = pltpu.async_copy(dm_buf, dstm_hbm.at[pl.ds(wrow, WPW)], sem_o)
    plsc.subcore_barrier()

    def hist_body(g, _):
        descs = [pltpu.async_copy(ones_buf, deg_sh.at[dm_buf.at[4 * g + b]],
                                  sem_i, add=True) for b in range(4)]
        for desc in descs:
            desc.wait()
        return 0

    lax.fori_loop(0, WPW // 4, hist_body, 0)
    out_s.wait()
    out_d.wait()
    plsc.subcore_barrier()
    dlo = pl.multiple_of(sid * RPTD, 16)
    pltpu.sync_copy(deg_sh.at[pl.ds(dlo, RPTD)],
                    degp_hbm.at[pl.ds(pl.multiple_of(cid * N_PADD + sid * RPTD,
                                                     16), RPTD)])


_sc_pre = pl.kernel(
    _sc_pre_body,
    out_type=(
        jax.ShapeDtypeStruct((NWIN, WIN), jnp.int32),
        jax.ShapeDtypeStruct((NWIN, WIN), jnp.int32),
        jax.ShapeDtypeStruct((NC * N_PADD,), jnp.float32),
    ),
    mesh=_mesh,
    scratch_types=[
        pltpu.VMEM((WPW, WIN), jnp.int32),
        pltpu.VMEM((WPW, WIN), jnp.int32),
        pltpu.VMEM((WPW, WIN), jnp.int32),
        pltpu.VMEM((WPW, WIN), jnp.int32),
        pltpu.VMEM((WIN,), jnp.float32),
        pltpu.VMEM((640,), jnp.float32),
        pltpu.SemaphoreType.DMA,
        pltpu.SemaphoreType.DMA,
        pltpu.VMEM_SHARED((N_PADD,), jnp.float32),
    ],
)


def _sc_spmm_body(hp_hbm, dstm_hbm, srcm_hbm, z_hbm, accp_hbm,
                  idx_d, idx_s, rows, sem_i, sem0, sem1, sem_a,
                  sem_s0, sem_s1, acc_sh):
    cid = lax.axis_index("c")
    sid = lax.axis_index("s")
    wid = cid * NS + sid
    wrow = pl.multiple_of(wid * WPW, WPW)
    sems = (sem0, sem1)

    def stage(c):
        cb = c % 2
        return (
            pltpu.async_copy(
                dstm_hbm.at[pl.ds(pl.multiple_of(wrow + c * CH, CH), CH)],
                idx_d.at[cb], sem_i),
            pltpu.async_copy(
                srcm_hbm.at[pl.ds(pl.multiple_of(wrow + c * CH, CH), CH)],
                idx_s.at[cb], sem_i),
        )

    def gather(k):
        return pltpu.async_copy(hp_hbm.at[idx_d.at[(k // CH) % 2, k % CH]],
                                rows.at[k % 2], sems[k % 2])

    st = {0: stage(0)}
    rlo = pl.multiple_of(sid * RPT, 8)

    @pl.when(cid == 0)
    def _():
        pltpu.async_copy(hp_hbm.at[pl.ds(rlo, RPT)],
                         acc_sh.at[pl.ds(rlo, RPT)], sem_a)

    @pl.when(cid != 0)
    def _():
        pltpu.async_copy(z_hbm, acc_sh.at[pl.ds(rlo, RPT)], sem_a)

    init = pltpu.make_async_copy(hp_hbm.at[pl.ds(rlo, RPT)],
                                 acc_sh.at[pl.ds(rlo, RPT)], sem_a)
    for d in st[0]:
        d.wait()

    ssems = (sem_s0, sem_s1)
    sd = {}

    def scatter(k):
        return pltpu.async_copy(rows.at[k % 2],
                                acc_sh.at[idx_s.at[(k // CH) % 2, k % CH]],
                                ssems[k % 2], add=True)

    gd = gather(0)
    for k in range(WPW):
        c = k // CH
        if k % CH == 0 and c + 1 < NCH:
            st[c + 1] = stage(c + 1)
        gnext = None
        if k + 1 < WPW:
            if (k + 1) % CH == 0:
                for d in st[(k + 1) // CH]:
                    d.wait()
            if k - 1 >= 0:
                sd[(k + 1) % 2].wait()
            gnext = gather(k + 1)
        gd.wait()
        if k == 0:
            init.wait()
            plsc.subcore_barrier()
        sd[k % 2] = scatter(k)
        gd = gnext

    sd[(WPW - 2) % 2].wait()
    sd[(WPW - 1) % 2].wait()
    plsc.subcore_barrier()
    pltpu.sync_copy(acc_sh.at[pl.ds(rlo, RPT)],
                    accp_hbm.at[pl.ds(pl.multiple_of(cid * N_PAD + sid * RPT, 8),
                                      RPT)])


_sc_spmm = pl.kernel(
    _sc_spmm_body,
    out_type=jax.ShapeDtypeStruct((NC * N_PAD, D), jnp.float32),
    mesh=_mesh,
    scratch_types=[
        pltpu.VMEM((2, CH, WIN), jnp.int32),
        pltpu.VMEM((2, CH, WIN), jnp.int32),
        pltpu.VMEM((2, WIN, D), jnp.float32),
        pltpu.SemaphoreType.DMA,
        pltpu.SemaphoreType.DMA,
        pltpu.SemaphoreType.DMA,
        pltpu.SemaphoreType.DMA,
        pltpu.SemaphoreType.DMA,
        pltpu.SemaphoreType.DMA,
        pltpu.VMEM_SHARED((N_PAD, D), jnp.float32),
    ],
)


def _dinv_from(degp):
    return lax.rsqrt(degp[0] + degp[1] + 1.0)


def _tc1_body(x_ref, w1_ref, b1_ref, degp_ref, out_ref):
    dinv = _dinv_from(degp_ref[...])
    h = jnp.dot(x_ref[...], w1_ref[...],
                preferred_element_type=jnp.float32) + b1_ref[...]
    out_ref[:N] = h * dinv[:N, None]
    out_ref[N:] = jnp.zeros((N_PAD - N, D), jnp.float32)


def _tc2_body(acc_ref, degp_ref, w2_ref, b2_ref, g_ref, be_ref,
              out_ref):
    dinv = _dinv_from(degp_ref[...])[:N, None]
    s = (acc_ref[0, :N] + acc_ref[1, :N]) * dinv
    mean = jnp.mean(s, axis=0)
    var = jnp.mean(jnp.square(s), axis=0) - jnp.square(mean)
    sn = (s - mean) * lax.rsqrt(var + BN_EPS) * g_ref[...] + be_ref[...]
    r = jnp.maximum(sn, 0.0)
    h2 = jnp.dot(r, w2_ref[...],
                 preferred_element_type=jnp.float32) + b2_ref[...]
    out_ref[:N] = h2 * dinv
    out_ref[N:] = jnp.zeros((N_PAD - N, D), jnp.float32)


def _tc3_body(acc_ref, degp_ref, out_ref):
    dinv = _dinv_from(degp_ref[...])[:N, None]
    out_ref[...] = (acc_ref[0, :N] + acc_ref[1, :N]) * dinv


_tc1 = pl.pallas_call(
    _tc1_body, out_shape=jax.ShapeDtypeStruct((N_PAD, D), jnp.float32))
_tc2 = pl.pallas_call(
    _tc2_body, out_shape=jax.ShapeDtypeStruct((N_PAD, D), jnp.float32))
_tc3 = pl.pallas_call(
    _tc3_body, out_shape=jax.ShapeDtypeStruct((N, D), jnp.float32))


def kernel(x, edge_index, W1, b1, W2, b2, gamma, beta):
    pad = jnp.zeros((E_PAD - E,), jnp.int32)
    src_in = jnp.concatenate([edge_index[0], pad]).reshape(NWIN, WIN)
    dst_in = jnp.concatenate([edge_index[1], pad]).reshape(NWIN, WIN)
    srcm, dstm, degp = _sc_pre(src_in, dst_in)
    degp = degp.reshape(NC, N_PADD)
    zz = jnp.zeros((RPT, D), jnp.float32)
    h1p = _tc1(x, W1, b1, degp)
    acc1 = _sc_spmm(h1p, dstm, srcm, zz).reshape(NC, N_PAD, D)
    h2p = _tc2(acc1, degp, W2, b2, gamma, beta)
    acc2 = _sc_spmm(h2p, dstm, srcm, zz).reshape(NC, N_PAD, D)
    return _tc3(acc2, degp)
